# Initial kernel scaffold; baseline (speedup 1.0000x reference)
import jax
import jax.numpy as jnp
from jax import lax
from jax.experimental import pallas as pl
from jax.experimental.pallas import tpu as pltpu
from jax.experimental.pallas import tpu_sc as plsc

IN_C = 16
OUT_C = 32
K = 3
B = 2
H = 16
W = 16
NA = 256
NJ = IN_C * K * K
HP = H + 2
WP = W + 2
ZN = B * HP * WP
ZNP = 656
L = 16
NX = B * IN_C * H * W


def _tec_body(x_hbm, bias_hbm, pc_hbm, out_hbm,
              pc_v, x_v, xi_v, lut_v, z_v, bias_v, out_v, sem):
  c = lax.axis_index("c")
  s = lax.axis_index("s")
  oc = s * 2 + c

  iota = lax.iota(jnp.int32, L)
  zeros_i = jnp.zeros((L,), jnp.int32)

  pc_copy = pltpu.async_copy(pc_hbm.at[oc], pc_v, sem)

  pltpu.sync_copy(x_hbm, x_v)
  pltpu.sync_copy(bias_hbm, bias_v)

  def quant_body(i, _):
    v = x_v[pl.ds(i * L, L)]
    q = (v * 255.0 + 0.5).astype(jnp.int32)
    xi_v[pl.ds(i * L, L)] = jnp.minimum(jnp.maximum(q, 0), 255)
    return 0
  lax.fori_loop(0, NX // L, quant_body, 0)

  pc_copy.wait()

  def chunk_body(ch, _):
    def row_body(j, accs):
      ae, ao = accs
      v = pc_v[j, pl.ds(ch * 2 * L, 2 * L)]
      e, o = plsc.unpack(v, format=plsc.PackFormat.INTERLEAVED)
      return ae + e, ao + o
    ae, ao = lax.fori_loop(0, NJ, row_body, (zeros_i, zeros_i))
    base = ch * 2 * L
    plsc.store_scatter(lut_v, [base + 2 * iota], ae)
    plsc.store_scatter(lut_v, [base + 1 + 2 * iota], ao)
    return 0
  lax.fori_loop(0, NA // (2 * L), chunk_body, 0)

  v0 = plsc.load_gather(lut_v, [zeros_i]) * IN_C
  def zinit_body(i, _):
    z_v[pl.ds(i * L, L)] = v0
    return 0
  lax.fori_loop(0, ZNP // L, zinit_body, 0)

  for b in range(B):
    def h_body(h, _):
      def ic_body(ic, acc):
        base = ((b * IN_C + ic) * H + h) * W
        return acc + plsc.load_gather(lut_v, [xi_v[pl.ds(base, L)]])
      acc = lax.fori_loop(0, IN_C, ic_body, zeros_i)
      zbase = (b * HP + h + 1) * WP + 1
      plsc.store_scatter(z_v, [zbase + iota], acc)
      return 0
    lax.fori_loop(0, H, h_body, 0)

  bias_g = plsc.load_gather(bias_v, [jnp.full((L,), oc, jnp.int32)])
  for b in range(B):
    def oh_body(oh, _):
      acc = zeros_i
      for dh in range(K):
        for dw in range(K):
          zbase = (b * HP + oh + dh) * WP + dw
          acc = acc + plsc.load_gather(z_v, [zbase + iota])
      out_v[b, oh, :] = acc.astype(jnp.float32) + bias_g
      return 0
    lax.fori_loop(0, H, oh_body, 0)

  for b in range(B):
    pltpu.sync_copy(out_v.at[b], out_hbm.at[b, oc])


@jax.jit
def _pcilt_conv(x_flat, bias, pc):
  f = pl.kernel(
      _tec_body,
      out_type=jax.ShapeDtypeStruct((B, OUT_C, H, W), jnp.float32),
      mesh=plsc.VectorSubcoreMesh(core_axis_name="c", subcore_axis_name="s"),
      scratch_types=[
          pltpu.VMEM((NJ, NA), jnp.int16),
          pltpu.VMEM((NX,), jnp.float32),
          pltpu.VMEM((NX,), jnp.int32),
          pltpu.VMEM((NA,), jnp.int32),
          pltpu.VMEM((ZNP,), jnp.int32),
          pltpu.VMEM((OUT_C,), jnp.float32),
          pltpu.VMEM((B, H, W), jnp.float32),
          pltpu.SemaphoreType.DMA,
      ],
  )
  return f(x_flat, bias, pc)


def kernel(x, weight, bias, pcilt):
  del weight
  pc = pcilt.reshape(OUT_C, NJ, NA)
  return _pcilt_conv(x.reshape(-1), bias, pc)


# speedup vs baseline: 9.0327x; 9.0327x over previous
import jax
import jax.numpy as jnp
from jax import lax
from jax.experimental import pallas as pl
from jax.experimental.pallas import tpu as pltpu
from jax.experimental.pallas import tpu_sc as plsc

IN_C = 16
OUT_C = 32
K = 3
B = 2
H = 16
W = 16
NA = 256
NJ = IN_C * K * K
HP = H + 2
WP = W + 2
ZN = B * HP * WP
ZNP = 656
L = 16
NX = B * IN_C * H * W


def _tec_body(x_hbm, bias_hbm, pc_hbm, out_hbm,
              pc_v, x_v, xi_v, lut_v, z_v, bias_v, out_v, sem):
  c = lax.axis_index("c")
  s = lax.axis_index("s")
  oc = s * 2 + c

  iota = lax.iota(jnp.int32, L)
  zeros_i = jnp.zeros((L,), jnp.int32)

  pc_copy = pltpu.async_copy(pc_hbm.at[oc], pc_v, sem)

  pltpu.sync_copy(x_hbm, x_v)
  pltpu.sync_copy(bias_hbm, bias_v)

  def quant_body(i, _):
    v = x_v[pl.ds(i * L, L)]
    q = (v * 255.0 + 0.5).astype(jnp.int32)
    xi_v[pl.ds(i * L, L)] = jnp.minimum(jnp.maximum(q, 0), 255)
    return 0
  lax.fori_loop(0, NX // L, quant_body, 0)

  pc_copy.wait()

  def reduce_chunk(ch):
    def row_body(j, accs):
      ae, ao = accs
      off = pl.multiple_of(j * NA + ch * 2 * L, 2 * L)
      v = pc_v[pl.ds(off, 2 * L)]
      w = plsc.bitcast(v, jnp.int32)
      e = lax.shift_right_arithmetic(lax.shift_left(w, 16), 16)
      o = lax.shift_right_arithmetic(w, 16)
      return ae + e, ao + o
    return lax.fori_loop(0, NJ, row_body, (zeros_i, zeros_i))

  def store_chunk(ch, ae, ao):
    base = ch * 2 * L
    plsc.store_scatter(lut_v, [base + 2 * iota], ae)
    plsc.store_scatter(lut_v, [base + 1 + 2 * iota], ao)

  ae0, ao0 = reduce_chunk(0)
  lut00 = jnp.sum(jnp.where(iota == 0, ae0, 0))
  store_chunk(0, ae0, ao0)

  def chunk_body(ch, _):
    ae, ao = reduce_chunk(ch)
    store_chunk(ch, ae, ao)
    return 0
  lax.fori_loop(1, NA // (2 * L), chunk_body, 0)

  v0 = jnp.full((L,), lut00 * IN_C, jnp.int32)
  def zinit_body(i, _):
    z_v[pl.ds(i * L, L)] = v0
    return 0
  lax.fori_loop(0, ZNP // L, zinit_body, 0)

  for b in range(B):
    def h_body(h, _):
      def ic_body(ic, acc):
        base = ((b * IN_C + ic) * H + h) * W
        return acc + plsc.load_gather(lut_v, [xi_v[pl.ds(base, L)]])
      acc = lax.fori_loop(0, IN_C, ic_body, zeros_i)
      zbase = (b * HP + h + 1) * WP + 1
      plsc.store_scatter(z_v, [zbase + iota], acc)
      return 0
    lax.fori_loop(0, H, h_body, 0)

  bias_g = plsc.load_gather(bias_v, [jnp.full((L,), oc, jnp.int32)])
  for b in range(B):
    def oh_body(oh, _):
      acc = zeros_i
      for dh in range(K):
        for dw in range(K):
          zbase = (b * HP + oh + dh) * WP + dw
          acc = acc + plsc.load_gather(z_v, [zbase + iota])
      out_v[b, oh, :] = acc.astype(jnp.float32) + bias_g
      return 0
    lax.fori_loop(0, H, oh_body, 0)

  for b in range(B):
    pltpu.sync_copy(out_v.at[b], out_hbm.at[b, oc])


@jax.jit
def _pcilt_conv(x_flat, bias, pc):
  f = pl.kernel(
      _tec_body,
      out_type=jax.ShapeDtypeStruct((B, OUT_C, H, W), jnp.float32),
      mesh=plsc.VectorSubcoreMesh(core_axis_name="c", subcore_axis_name="s"),
      compiler_params=pltpu.CompilerParams(
          use_tc_tiling_on_sc=False, needs_layout_passes=False),
      scratch_types=[
          pltpu.VMEM((NJ * NA,), jnp.int16),
          pltpu.VMEM((NX,), jnp.float32),
          pltpu.VMEM((NX,), jnp.int32),
          pltpu.VMEM((NA,), jnp.int32),
          pltpu.VMEM((ZNP,), jnp.int32),
          pltpu.VMEM((OUT_C,), jnp.float32),
          pltpu.VMEM((B, H, W), jnp.float32),
          pltpu.SemaphoreType.DMA,
      ],
  )
  return f(x_flat, bias, pc)


def kernel(x, weight, bias, pcilt):
  del weight
  pc = pcilt.reshape(OUT_C, NJ * NA)
  return _pcilt_conv(x.reshape(-1), bias, pc)


# speedup vs baseline: 10.1562x; 1.1244x over previous
import jax
import jax.numpy as jnp
from jax import lax
from jax.experimental import pallas as pl
from jax.experimental.pallas import tpu as pltpu
from jax.experimental.pallas import tpu_sc as plsc

IN_C = 16
OUT_C = 32
K = 3
B = 2
H = 16
W = 16
NA = 256
NJ = IN_C * K * K
HP = H + 2
WP = W + 2
ZN = B * HP * WP
ZNP = 656
L = 16
NX = B * IN_C * H * W


def _tec_body(x_hbm, bias_hbm, pc_hbm, out_hbm,
              pc_v, x_v, xi_v, lut_v, z_v, bias_v, out_v, sem):
  c = lax.axis_index("c")
  s = lax.axis_index("s")
  oc = s * 2 + c

  iota = lax.iota(jnp.int32, L)
  zeros_i = jnp.zeros((L,), jnp.int32)

  pc_copy = pltpu.async_copy(pc_hbm.at[oc], pc_v, sem)

  pltpu.sync_copy(x_hbm, x_v)
  pltpu.sync_copy(bias_hbm, bias_v)

  QU = 4
  def quant_body(i, _):
    for u in range(QU):
      off = pl.multiple_of(i * QU * L + u * L, L)
      v = x_v[pl.ds(off, L)]
      xi_v[pl.ds(off, L)] = (v * 255.0 + 0.5).astype(jnp.int32)
    return 0
  lax.fori_loop(0, NX // (QU * L), quant_body, 0)

  pc_copy.wait()

  NCH = NA // (2 * L)
  def row_body(j, accs):
    base = pl.multiple_of(j * NA, NA)
    out = []
    for ch in range(NCH):
      v = pc_v[pl.ds(base + ch * 2 * L, 2 * L)]
      w = plsc.bitcast(v, jnp.int32)
      e = lax.shift_right_arithmetic(lax.shift_left(w, 16), 16)
      o = lax.shift_right_arithmetic(w, 16)
      out.append(accs[2 * ch] + e)
      out.append(accs[2 * ch + 1] + o)
    return tuple(out)
  accs = lax.fori_loop(0, NJ, row_body, (zeros_i,) * (2 * NCH))

  lut00 = jnp.sum(jnp.where(iota == 0, accs[0], 0))
  for ch in range(NCH):
    base = ch * 2 * L
    plsc.store_scatter(lut_v, [base + 2 * iota], accs[2 * ch])
    plsc.store_scatter(lut_v, [base + 1 + 2 * iota], accs[2 * ch + 1])

  v0 = jnp.full((L,), lut00 * IN_C, jnp.int32)
  def zinit_body(i, _):
    z_v[pl.ds(i * L, L)] = v0
    return 0
  lax.fori_loop(0, ZNP // L, zinit_body, 0)

  for b in range(B):
    def h_body(h, _):
      hbase = pl.multiple_of((b * IN_C * H + h) * W, L)
      acc = zeros_i
      for ic in range(IN_C):
        idx = xi_v[pl.ds(hbase + ic * H * W, L)]
        acc = acc + plsc.load_gather(lut_v, [idx])
      zbase = (b * HP + h + 1) * WP + 1
      plsc.store_scatter(z_v, [zbase + iota], acc)
      return 0
    lax.fori_loop(0, H, h_body, 0)

  bias_g = plsc.load_gather(bias_v, [jnp.full((L,), oc, jnp.int32)])
  for b in range(B):
    def oh_body(oh, _):
      acc = zeros_i
      for dh in range(K):
        for dw in range(K):
          zbase = (b * HP + oh + dh) * WP + dw
          acc = acc + plsc.load_gather(z_v, [zbase + iota])
      out_v[b, oh, :] = acc.astype(jnp.float32) + bias_g
      return 0
    lax.fori_loop(0, H, oh_body, 0)

  for b in range(B):
    pltpu.sync_copy(out_v.at[b], out_hbm.at[b, oc])


@jax.jit
def _pcilt_conv(x_flat, bias, pc):
  f = pl.kernel(
      _tec_body,
      out_type=jax.ShapeDtypeStruct((B, OUT_C, H, W), jnp.float32),
      mesh=plsc.VectorSubcoreMesh(core_axis_name="c", subcore_axis_name="s"),
      compiler_params=pltpu.CompilerParams(
          use_tc_tiling_on_sc=False, needs_layout_passes=False),
      scratch_types=[
          pltpu.VMEM((NJ * NA,), jnp.int16),
          pltpu.VMEM((NX,), jnp.float32),
          pltpu.VMEM((NX,), jnp.int32),
          pltpu.VMEM((NA,), jnp.int32),
          pltpu.VMEM((ZNP,), jnp.int32),
          pltpu.VMEM((OUT_C,), jnp.float32),
          pltpu.VMEM((B, H, W), jnp.float32),
          pltpu.SemaphoreType.DMA,
      ],
  )
  return f(x_flat, bias, pc)


def kernel(x, weight, bias, pcilt):
  del weight
  pc = pcilt.reshape(OUT_C, NJ * NA)
  return _pcilt_conv(x.reshape(-1), bias, pc)


# speedup vs baseline: 12.8686x; 1.2671x over previous
import jax
import jax.numpy as jnp
from jax import lax
from jax.experimental import pallas as pl
from jax.experimental.pallas import tpu as pltpu
from jax.experimental.pallas import tpu_sc as plsc

IN_C = 16
OUT_C = 32
K = 3
B = 2
H = 16
W = 16
NA = 256
NJ = IN_C * K * K
HP = H + 2
WP = W + 2
ZNP = 656
L = 16
NX = B * IN_C * H * W


def _tec_body(x_hbm, bias_hbm, pc_hbm, out_hbm,
              pc_v, x_v, xi_sl, xi_v, lut_v, z_v, bias_v, out_v, xi_sh,
              sem0, sem1):
  c = lax.axis_index("c")
  s = lax.axis_index("s")
  oc = s * 2 + c

  iota = lax.iota(jnp.int32, L)
  zeros_i = jnp.zeros((L,), jnp.int32)

  NHALF = NJ * NA // 2
  pc_copy0 = pltpu.async_copy(
      pc_hbm.at[oc, 0], pc_v.at[pl.ds(0, NHALF)], sem0)
  pc_copy1 = pltpu.async_copy(
      pc_hbm.at[oc, 1], pc_v.at[pl.ds(NHALF, NHALF)], sem1)

  NSL = NX // 16
  QU = 4
  pltpu.sync_copy(x_hbm.at[pl.ds(s * NSL, NSL)], x_v)
  def quant_body(i, _):
    for u in range(QU):
      off = pl.multiple_of(i * QU * L + u * L, L)
      v = x_v[pl.ds(off, L)]
      xi_sl[pl.ds(off, L)] = (v * 255.0 + 0.5).astype(jnp.int32)
    return 0
  lax.fori_loop(0, NSL // (QU * L), quant_body, 0)
  pltpu.sync_copy(xi_sl, xi_sh.at[pl.ds(s * NSL, NSL)])
  pltpu.sync_copy(bias_hbm, bias_v)
  plsc.subcore_barrier()
  pltpu.sync_copy(xi_sh, xi_v)

  NCH = NA // (2 * L)
  def row_body(j, accs):
    base = pl.multiple_of(j * NA, NA)
    out = []
    for ch in range(NCH):
      v = pc_v[pl.ds(base + ch * 2 * L, 2 * L)]
      w = plsc.bitcast(v, jnp.int32)
      e = lax.shift_right_arithmetic(lax.shift_left(w, 16), 16)
      o = lax.shift_right_arithmetic(w, 16)
      out.append(accs[2 * ch] + e)
      out.append(accs[2 * ch + 1] + o)
    return tuple(out)
  pc_copy0.wait()
  accs = lax.fori_loop(0, NJ // 2, row_body, (zeros_i,) * (2 * NCH))
  pc_copy1.wait()
  accs = lax.fori_loop(NJ // 2, NJ, row_body, accs)

  lut00 = jnp.sum(jnp.where(iota == 0, accs[0], 0))
  for ch in range(NCH):
    base = ch * 2 * L
    plsc.store_scatter(lut_v, [base + 2 * iota], accs[2 * ch])
    plsc.store_scatter(lut_v, [base + 1 + 2 * iota], accs[2 * ch + 1])

  v0 = jnp.full((L,), lut00 * IN_C, jnp.int32)
  def zinit_body(i, _):
    z_v[pl.ds(i * L, L)] = v0
    return 0
  lax.fori_loop(0, ZNP // L, zinit_body, 0)

  for b in range(B):
    def h_body(h, _):
      hbase = pl.multiple_of((b * IN_C * H + h) * W, L)
      acc = zeros_i
      for ic in range(IN_C):
        idx = xi_v[pl.ds(hbase + ic * H * W, L)]
        acc = acc + plsc.load_gather(lut_v, [idx])
      zbase = (b * HP + h + 1) * WP + 1
      plsc.store_scatter(z_v, [zbase + iota], acc)
      return 0
    lax.fori_loop(0, H, h_body, 0)

  bias_g = plsc.load_gather(bias_v, [jnp.full((L,), oc, jnp.int32)])
  for b in range(B):
    def oh_body(oh, _):
      acc = zeros_i
      for dh in range(K):
        for dw in range(K):
          zbase = (b * HP + oh + dh) * WP + dw
          acc = acc + plsc.load_gather(z_v, [zbase + iota])
      out_v[b, oh, :] = acc.astype(jnp.float32) + bias_g
      return 0
    lax.fori_loop(0, H, oh_body, 0)

  for b in range(B):
    pltpu.sync_copy(out_v.at[b], out_hbm.at[b, oc])


@jax.jit
def _pcilt_conv(x_flat, bias, pc):
  f = pl.kernel(
      _tec_body,
      out_type=jax.ShapeDtypeStruct((B, OUT_C, H, W), jnp.float32),
      mesh=plsc.VectorSubcoreMesh(core_axis_name="c", subcore_axis_name="s"),
      compiler_params=pltpu.CompilerParams(
          use_tc_tiling_on_sc=False, needs_layout_passes=False),
      scratch_types=[
          pltpu.VMEM((NJ * NA,), jnp.int16),
          pltpu.VMEM((NX // 16,), jnp.float32),
          pltpu.VMEM((NX // 16,), jnp.int32),
          pltpu.VMEM((NX,), jnp.int32),
          pltpu.VMEM((NA,), jnp.int32),
          pltpu.VMEM((ZNP,), jnp.int32),
          pltpu.VMEM((OUT_C,), jnp.float32),
          pltpu.VMEM((B, H, W), jnp.float32),
          pltpu.VMEM_SHARED((NX,), jnp.int32),
          pltpu.SemaphoreType.DMA,
          pltpu.SemaphoreType.DMA,
      ],
  )
  return f(x_flat, bias, pc)


def kernel(x, weight, bias, pcilt):
  del weight
  pc = pcilt.transpose(0, 2, 3, 1, 4).reshape(OUT_C, 2, NJ * NA // 2)
  return _pcilt_conv(x.reshape(-1), bias, pc)
